# baseline (speedup 1.0000x reference)
import functools

import jax
import jax.numpy as jnp
from jax import lax
from jax.experimental import pallas as pl
from jax.experimental.pallas import tpu as pltpu
from jax.experimental.pallas import tpu_sc as plsc

N_ROWS = 100000
DIM = 256
G = 1024
NC = 2
NS = 16
L = 16
NW = NC * NS
GPW = G // NW
CH = 128
NCOL = DIM // L
BPAD = 48


def _sc_body(x_hbm, batch_hbm, mask_hbm, w_hbm, b_hbm, bounds_hbm, out_hbm,
             xbuf, bidx, midx, wbuf, bbuf, boundsbuf, accbuf,
             endbuf, dendbuf, flagbuf, accmem, denmem, sem0, sem1):
    cid = lax.axis_index("c")
    sid = lax.axis_index("s")
    wid = cid * NS + sid
    iot = lax.iota(jnp.int32, L)
    zero16 = jnp.zeros((L,), jnp.float32)
    one16 = jnp.ones((L,), jnp.float32)

    pltpu.sync_copy(w_hbm, wbuf)
    pltpu.sync_copy(b_hbm, bbuf)
    pltpu.sync_copy(bounds_hbm, boundsbuf)

    def _zflag(r, _):
        flagbuf[r, pl.ds(0, L)] = zero16
        return 0
    lax.fori_loop(0, GPW, _zflag, 0)
    for cc in range(NCOL):
        accmem[pl.ds(cc * L, L)] = zero16
    denmem[pl.ds(0, L)] = zero16

    def _scalar_at(ref, i):
        return plsc.load_gather(ref, [jnp.full((L,), i, jnp.int32)])[0]

    r_lo = _scalar_at(boundsbuf, wid)
    r_hi = _scalar_at(boundsbuf, wid + 1)
    start0 = (r_lo // CH) * CH
    nchp = jnp.maximum((r_hi - start0 + CH - 1) // CH, 0)

    wregs = [wbuf[pl.ds(cc * L, L)] for cc in range(NCOL)]
    b_s = bbuf[...][0]
    g_base = wid * GPW

    def _chunk_start(k):
        return jnp.minimum(start0 + k * CH, N_ROWS - CH)

    def _copies(k, bsl, sem):
        s = _chunk_start(k)
        return (pltpu.make_async_copy(x_hbm.at[pl.ds(s, CH)],
                                      xbuf.at[bsl], sem),
                pltpu.make_async_copy(batch_hbm.at[pl.ds(s, CH)],
                                      bidx.at[bsl], sem),
                pltpu.make_async_copy(mask_hbm.at[pl.ds(s, CH)],
                                      midx.at[bsl], sem))

    def _start(k, bsl, sem):
        for d in _copies(k, bsl, sem):
            d.start()

    def _wait(k, bsl, sem):
        for d in _copies(k, bsl, sem):
            d.wait()

    def _flush(cur_g):
        gl = jnp.clip(cur_g - g_base, 0, GPW - 1)

        def fbody(i, _):
            endbuf[gl, pl.ds(i * L, L)] = accmem[pl.ds(i * L, L)]
            return 0
        lax.fori_loop(0, NCOL, fbody, 0)
        dendbuf[gl, pl.ds(0, L)] = denmem[pl.ds(0, L)]
        flagbuf[gl, pl.ds(0, L)] = one16

    def _process(k, bsl, cur_g0):
        xb = xbuf.at[bsl]
        bb = bidx.at[bsl]
        mb = midx.at[bsl]
        s = _chunk_start(k)
        base = start0 + k * CH
        lo = jnp.maximum(r_lo, base)

        def group_body(rg, cur_g):
            r0 = rg * L
            bv = bb[pl.ds(r0, L)]
            mv = mb[pl.ds(r0, L)]
            gidv = (s + r0) + iot
            validv = (gidv >= lo) & (gidv < r_hi)
            selm = jnp.where(validv & (mv == 1), 1.0, 0.0)
            anysw = jnp.any((bv != cur_g) & validv)

            zacc = zero16
            for j in range(L):
                r = r0 + j
                dv = zero16
                for cc in range(0, NCOL, 2):
                    dv = dv + (xb[r, pl.ds(cc * L, L)] * wregs[cc]
                               + xb[r, pl.ds((cc + 1) * L, L)]
                               * wregs[cc + 1])
                z = jnp.sum(dv)
                zacc = jnp.where(iot == j, jnp.full((L,), z, jnp.float32),
                                 zacc)
            u = jnp.exp(-(zacc + b_s))
            gate = 1.0 / (1.0 + u)
            evv = jnp.exp(gate) * selm

            def fast(cur_g):
                accs = [zero16] * NCOL
                for j in range(L):
                    r = r0 + j
                    eb = jnp.full((L,), evv[j], jnp.float32)
                    for cc in range(NCOL):
                        accs[cc] = accs[cc] + xb[r, pl.ds(cc * L, L)] * eb
                for cc in range(NCOL):
                    accmem[pl.ds(cc * L, L)] = (accmem[pl.ds(cc * L, L)]
                                                + accs[cc])
                denmem[pl.ds(0, L)] = denmem[pl.ds(0, L)] + evv
                return cur_g

            def slow(cur_g):
                _flush(cur_g)

                cg = cur_g
                for j in range(L):
                    gid = s + r0 + j
                    valid = (gid >= lo) & (gid < r_hi)
                    b_r = bv[j]
                    slot = jnp.clip(b_r - g_base, 0, GPW - 1)
                    eb = jnp.full((L,), evv[j], jnp.float32)
                    dnew = denmem[pl.ds(0, L)] + jnp.where(iot == j, eb,
                                                           zero16)
                    denmem[pl.ds(0, L)] = dnew
                    dendbuf[slot, pl.ds(0, L)] = dnew

                    for i in range(NCOL):
                        anew = (accmem[pl.ds(i * L, L)]
                                + xb[r0 + j, pl.ds(i * L, L)] * eb)
                        accmem[pl.ds(i * L, L)] = anew
                        endbuf[slot, pl.ds(i * L, L)] = anew
                    flagbuf[slot, pl.ds(0, L)] = one16
                    cg = jnp.where(valid, b_r, cg)
                return cg

            return lax.cond(anysw, slow, fast, cur_g)

        return lax.fori_loop(0, CH // L, group_body, cur_g0)

    @pl.when(nchp > 0)
    def _():
        _start(0, 0, sem0)

    def chunk_body(k, cur_g):
        bsl = k % 2

        @pl.when((k + 1 < nchp) & (bsl == 0))
        def _():
            _start(k + 1, 1, sem1)

        @pl.when((k + 1 < nchp) & (bsl == 1))
        def _():
            _start(k + 1, 0, sem0)

        @pl.when(bsl == 0)
        def _():
            _wait(k, 0, sem0)

        @pl.when(bsl == 1)
        def _():
            _wait(k, 1, sem1)

        return _process(k, bsl, cur_g)

    cur_g = lax.fori_loop(0, nchp, chunk_body, jnp.int32(-1))

    @pl.when(cur_g >= 0)
    def _():
        _flush(cur_g)

    def post_body(slot, c2):
        prev_den = c2[0]
        prevs = c2[1:]
        fv = flagbuf[slot, pl.ds(0, L)] > 0.5
        den_eff = jnp.where(fv, dendbuf[slot, pl.ds(0, L)], prev_den)
        totv = jnp.full((L,), jnp.sum(den_eff - prev_den), jnp.float32)
        rec = 1.0 / jnp.where(totv > 0.0, totv, one16)
        new_prevs = []
        for cc in range(NCOL):
            ecur = jnp.where(fv, endbuf[slot, pl.ds(cc * L, L)], prevs[cc])
            accbuf[slot, pl.ds(cc * L, L)] = (ecur - prevs[cc]) * rec
            new_prevs.append(ecur)
        return (den_eff,) + tuple(new_prevs)

    lax.fori_loop(0, GPW, post_body,
                  (zero16,) + tuple(zero16 for _ in range(NCOL)))

    pltpu.sync_copy(accbuf, out_hbm.at[pl.ds(g_base, GPW)])


_sc_pool = functools.partial(
    pl.kernel,
    out_type=jax.ShapeDtypeStruct((G, DIM), jnp.float32),
    mesh=plsc.VectorSubcoreMesh(core_axis_name="c", subcore_axis_name="s"),
    scratch_types=[
        pltpu.VMEM((2, CH, DIM), jnp.float32),
        pltpu.VMEM((2, CH), jnp.int32),
        pltpu.VMEM((2, CH), jnp.int32),
        pltpu.VMEM((DIM,), jnp.float32),
        pltpu.VMEM((L,), jnp.float32),
        pltpu.VMEM((BPAD,), jnp.int32),
        pltpu.VMEM((GPW, DIM), jnp.float32),
        pltpu.VMEM((GPW, DIM), jnp.float32),
        pltpu.VMEM((GPW, L), jnp.float32),
        pltpu.VMEM((GPW, L), jnp.float32),
        pltpu.VMEM((DIM,), jnp.float32),
        pltpu.VMEM((L,), jnp.float32),
        pltpu.SemaphoreType.DMA,
        pltpu.SemaphoreType.DMA,
    ],
    compiler_params=pltpu.CompilerParams(needs_layout_passes=False),
)(_sc_body)


@jax.jit
def kernel(x, batch, repeat_unit_mask, gate_w, gate_b):
    batch = batch.astype(jnp.int32)
    w = gate_w.reshape(DIM).astype(jnp.float32)
    b = jnp.broadcast_to(gate_b.astype(jnp.float32), (L,))
    edges = jnp.arange(0, G + GPW, GPW, dtype=jnp.int32)
    bounds = jnp.searchsorted(batch, edges).astype(jnp.int32)
    bounds = jnp.pad(bounds, (0, BPAD - bounds.shape[0]))
    return _sc_pool(x, batch, repeat_unit_mask.astype(jnp.int32), w, b,
                    bounds)


# speedup vs baseline: 1.5021x; 1.5021x over previous
import functools

import jax
import jax.numpy as jnp
from jax import lax
from jax.experimental import pallas as pl
from jax.experimental.pallas import tpu as pltpu
from jax.experimental.pallas import tpu_sc as plsc

N_ROWS = 100000
DIM = 256
G = 1024
NC = 2
NS = 16
L = 16
NW = NC * NS
GPW = G // NW
CH = 128
NCOL = DIM // L
BPAD = 48


def _sc_body(x_hbm, batch_hbm, mask_hbm, w_hbm, b_hbm, bounds_hbm, out_hbm,
             xbuf, bidx, midx, wbuf, bbuf, boundsbuf, accbuf,
             endbuf, dendbuf, flagbuf, accmem, denmem, sem0, sem1):
    cid = lax.axis_index("c")
    sid = lax.axis_index("s")
    wid = cid * NS + sid
    iot = lax.iota(jnp.int32, L)
    zero16 = jnp.zeros((L,), jnp.float32)
    one16 = jnp.ones((L,), jnp.float32)

    pltpu.sync_copy(w_hbm, wbuf)
    pltpu.sync_copy(b_hbm, bbuf)
    pltpu.sync_copy(bounds_hbm, boundsbuf)

    def _zflag(r, _):
        flagbuf[r, pl.ds(0, L)] = zero16
        return 0
    lax.fori_loop(0, GPW, _zflag, 0)
    for cc in range(NCOL):
        accmem[pl.ds(cc * L, L)] = zero16
    denmem[pl.ds(0, L)] = zero16

    def _scalar_at(ref, i):
        return plsc.load_gather(ref, [jnp.full((L,), i, jnp.int32)])[0]

    r_lo = _scalar_at(boundsbuf, wid)
    r_hi = _scalar_at(boundsbuf, wid + 1)
    start0 = (r_lo // CH) * CH
    nchp = jnp.maximum((r_hi - start0 + CH - 1) // CH, 0)

    wregs = [wbuf[pl.ds(cc * L, L)] for cc in range(NCOL)]
    b_s = bbuf[...][0]
    g_base = wid * GPW

    def _chunk_start(k):
        return jnp.minimum(start0 + k * CH, N_ROWS - CH)

    def _copies(k, bsl, sem):
        s = _chunk_start(k)
        return (pltpu.make_async_copy(x_hbm.at[pl.ds(s, CH)],
                                      xbuf.at[bsl], sem),
                pltpu.make_async_copy(batch_hbm.at[pl.ds(s, CH)],
                                      bidx.at[bsl], sem),
                pltpu.make_async_copy(mask_hbm.at[pl.ds(s, CH)],
                                      midx.at[bsl], sem))

    def _start(k, bsl, sem):
        for d in _copies(k, bsl, sem):
            d.start()

    def _wait(k, bsl, sem):
        for d in _copies(k, bsl, sem):
            d.wait()

    def _flush(cur_g):
        gl = jnp.clip(cur_g - g_base, 0, GPW - 1)

        def fbody(i, _):
            endbuf[gl, pl.ds(i * L, L)] = accmem[pl.ds(i * L, L)]
            return 0
        lax.fori_loop(0, NCOL, fbody, 0)
        dendbuf[gl, pl.ds(0, L)] = denmem[pl.ds(0, L)]
        flagbuf[gl, pl.ds(0, L)] = one16

    def _process(k, bsl, cur_g0):
        xb = xbuf.at[bsl]
        bb = bidx.at[bsl]
        mb = midx.at[bsl]
        s = _chunk_start(k)
        base = start0 + k * CH
        lo = jnp.maximum(r_lo, base)

        def group_body(rg, cur_g):
            r0 = rg * L
            bv = bb[pl.ds(r0, L)]
            mv = mb[pl.ds(r0, L)]
            gidv = (s + r0) + iot
            validv = (gidv >= lo) & (gidv < r_hi)
            selm = jnp.where(validv & (mv == 1), 1.0, 0.0)
            anysw = jnp.any((bv != cur_g) & validv)

            def arow(jj, zacc):
                ra = r0 + 2 * jj
                dv0 = zero16
                dv1 = zero16
                for cc in range(0, NCOL, 2):
                    dv0 = dv0 + (xb[ra, pl.ds(cc * L, L)] * wregs[cc]
                                 + xb[ra, pl.ds((cc + 1) * L, L)]
                                 * wregs[cc + 1])
                    dv1 = dv1 + (xb[ra + 1, pl.ds(cc * L, L)] * wregs[cc]
                                 + xb[ra + 1, pl.ds((cc + 1) * L, L)]
                                 * wregs[cc + 1])
                z0 = jnp.full((L,), jnp.sum(dv0), jnp.float32)
                z1 = jnp.full((L,), jnp.sum(dv1), jnp.float32)
                zacc = jnp.where(iot == 2 * jj, z0, zacc)
                return jnp.where(iot == 2 * jj + 1, z1, zacc)

            zacc = lax.fori_loop(0, L // 2, arow, zero16)
            u = jnp.exp(-(zacc + b_s))
            gate = 1.0 / (1.0 + u)
            evv = jnp.exp(gate) * selm

            def fast(cur_g):
                def frow(jj, accs):
                    ra = r0 + 4 * jj
                    ebs = [jnp.take(evv,
                                    jnp.full((L,), 4 * jj + q, jnp.int32))
                           for q in range(4)]
                    return tuple(
                        accs[cc]
                        + xb[ra, pl.ds(cc * L, L)] * ebs[0]
                        + xb[ra + 1, pl.ds(cc * L, L)] * ebs[1]
                        + xb[ra + 2, pl.ds(cc * L, L)] * ebs[2]
                        + xb[ra + 3, pl.ds(cc * L, L)] * ebs[3]
                        for cc in range(NCOL))

                accs = lax.fori_loop(0, 4, frow,
                                     tuple(zero16 for _ in range(NCOL)))
                for cc in range(NCOL):
                    accmem[pl.ds(cc * L, L)] = (accmem[pl.ds(cc * L, L)]
                                                + accs[cc])
                denmem[pl.ds(0, L)] = denmem[pl.ds(0, L)] + evv
                return cur_g

            def slow(cur_g):
                _flush(cur_g)

                def rbody(j, cg):
                    gid = s + r0 + j
                    valid = (gid >= lo) & (gid < r_hi)
                    jf = jnp.full((L,), j, jnp.int32)
                    b_r = jnp.take(bv, jf)[0]
                    slot = jnp.clip(b_r - g_base, 0, GPW - 1)
                    eb = jnp.take(evv, jf)
                    dnew = denmem[pl.ds(0, L)] + jnp.where(iot == j, eb,
                                                           zero16)
                    denmem[pl.ds(0, L)] = dnew
                    dendbuf[slot, pl.ds(0, L)] = dnew

                    def cbody(i, _):
                        anew = (accmem[pl.ds(i * L, L)]
                                + xb[r0 + j, pl.ds(i * L, L)] * eb)
                        accmem[pl.ds(i * L, L)] = anew
                        endbuf[slot, pl.ds(i * L, L)] = anew
                        return 0
                    lax.fori_loop(0, NCOL, cbody, 0)
                    flagbuf[slot, pl.ds(0, L)] = one16
                    return jnp.where(valid, b_r, cg)

                return lax.fori_loop(0, L, rbody, cur_g)

            return lax.cond(anysw, slow, fast, cur_g)

        return lax.fori_loop(0, CH // L, group_body, cur_g0)

    @pl.when(nchp > 0)
    def _():
        _start(0, 0, sem0)

    def chunk_body(k, cur_g):
        bsl = k % 2

        @pl.when((k + 1 < nchp) & (bsl == 0))
        def _():
            _start(k + 1, 1, sem1)

        @pl.when((k + 1 < nchp) & (bsl == 1))
        def _():
            _start(k + 1, 0, sem0)

        @pl.when(bsl == 0)
        def _():
            _wait(k, 0, sem0)

        @pl.when(bsl == 1)
        def _():
            _wait(k, 1, sem1)

        return _process(k, bsl, cur_g)

    cur_g = lax.fori_loop(0, nchp, chunk_body, jnp.int32(-1))

    @pl.when(cur_g >= 0)
    def _():
        _flush(cur_g)

    def post_body(slot, c2):
        prev_den = c2[0]
        prevs = c2[1:]
        fv = flagbuf[slot, pl.ds(0, L)] > 0.5
        den_eff = jnp.where(fv, dendbuf[slot, pl.ds(0, L)], prev_den)
        totv = jnp.full((L,), jnp.sum(den_eff - prev_den), jnp.float32)
        rec = 1.0 / jnp.where(totv > 0.0, totv, one16)
        new_prevs = []
        for cc in range(NCOL):
            ecur = jnp.where(fv, endbuf[slot, pl.ds(cc * L, L)], prevs[cc])
            accbuf[slot, pl.ds(cc * L, L)] = (ecur - prevs[cc]) * rec
            new_prevs.append(ecur)
        return (den_eff,) + tuple(new_prevs)

    lax.fori_loop(0, GPW, post_body,
                  (zero16,) + tuple(zero16 for _ in range(NCOL)))

    pltpu.sync_copy(accbuf, out_hbm.at[pl.ds(g_base, GPW)])


_sc_pool = functools.partial(
    pl.kernel,
    out_type=jax.ShapeDtypeStruct((G, DIM), jnp.float32),
    mesh=plsc.VectorSubcoreMesh(core_axis_name="c", subcore_axis_name="s"),
    scratch_types=[
        pltpu.VMEM((2, CH, DIM), jnp.float32),
        pltpu.VMEM((2, CH), jnp.int32),
        pltpu.VMEM((2, CH), jnp.int32),
        pltpu.VMEM((DIM,), jnp.float32),
        pltpu.VMEM((L,), jnp.float32),
        pltpu.VMEM((BPAD,), jnp.int32),
        pltpu.VMEM((GPW, DIM), jnp.float32),
        pltpu.VMEM((GPW, DIM), jnp.float32),
        pltpu.VMEM((GPW, L), jnp.float32),
        pltpu.VMEM((GPW, L), jnp.float32),
        pltpu.VMEM((DIM,), jnp.float32),
        pltpu.VMEM((L,), jnp.float32),
        pltpu.SemaphoreType.DMA,
        pltpu.SemaphoreType.DMA,
    ],
    compiler_params=pltpu.CompilerParams(needs_layout_passes=False),
)(_sc_body)


@jax.jit
def kernel(x, batch, repeat_unit_mask, gate_w, gate_b):
    batch = batch.astype(jnp.int32)
    w = gate_w.reshape(DIM).astype(jnp.float32)
    b = jnp.broadcast_to(gate_b.astype(jnp.float32), (L,))
    edges = jnp.arange(0, G + GPW, GPW, dtype=jnp.int32)
    bounds = jnp.searchsorted(batch, edges).astype(jnp.int32)
    bounds = jnp.pad(bounds, (0, BPAD - bounds.shape[0]))
    return _sc_pool(x, batch, repeat_unit_mask.astype(jnp.int32), w, b,
                    bounds)
